# baseline (speedup 1.0000x reference)
import functools

import jax
import jax.numpy as jnp
from jax import lax
from jax.experimental import pallas as pl
from jax.experimental.pallas import tpu as pltpu
from jax.experimental.pallas import tpu_sc as plsc

_NC = 2
_NS = 16
_NW = _NC * _NS
_NB = 2


def _embed_sc(tokg, w2, B, S, D):
    G = tokg.shape[0]
    blocks_per_w = G // _NW
    niter = blocks_per_w // _NB

    mesh = plsc.VectorSubcoreMesh(core_axis_name="c", subcore_axis_name="s")

    @functools.partial(
        pl.kernel,
        mesh=mesh,
        compiler_params=pltpu.CompilerParams(needs_layout_passes=False),
        out_type=jax.ShapeDtypeStruct((S, D, B), jnp.float32),
        scratch_types=[
            pltpu.VMEM((blocks_per_w, 128), jnp.int32),
            pltpu.VMEM((_NB, 128), jnp.int32),
            pltpu.VMEM((_NB, 128), jnp.int32),
            pltpu.VMEM((_NB, 128, 128), jnp.float32),
            pltpu.VMEM((_NB, D, 128), jnp.float32),
            pltpu.SemaphoreType.DMA((_NB,)),
            pltpu.SemaphoreType.DMA((_NB,)),
        ],
    )
    def emb(tok_hbm, w_hbm, out_hbm, idx_all, pr_idx, off_all, g_pair,
            o_buf, gsem, osem):
        wid = lax.axis_index("s") * _NC + lax.axis_index("c")
        base = wid * blocks_per_w

        pltpu.sync_copy(tok_hbm.at[pl.ds(base, blocks_per_w)], idx_all)

        def prep(l, b):
            for v in range(8):
                tid = idx_all[l, pl.ds(16 * v, 16)]
                pr_idx[b, pl.ds(16 * v, 16)] = tid >> 1
                off_all[b, pl.ds(16 * v, 16)] = (tid & 1) * 64

        def fire_gather(l, b):
            pltpu.async_copy(
                w_hbm.at[pr_idx.at[b]], g_pair.at[b], gsem.at[b])

        def drain_gather(l, b):
            pltpu.make_async_copy(
                w_hbm.at[pr_idx.at[b]], g_pair.at[b], gsem.at[b]).wait()

        def select(b):
            iot = lax.iota(jnp.int32, 16)
            for j in range(8):
                rows_v = iot + (16 * j)
                off_v = off_all[b, pl.ds(16 * j, 16)]
                for d in range(D):
                    o_buf[b, d, pl.ds(16 * j, 16)] = plsc.load_gather(
                        g_pair.at[b], [rows_v, off_v + d])

        def out_ref(l, b):
            g = base + l
            s_idx = g >> 7
            bg = g & 127
            return out_hbm.at[s_idx, :, pl.ds(bg * 128, 128)]

        def fire_out(l, b):
            pltpu.async_copy(o_buf.at[b], out_ref(l, b), osem.at[b])

        def wait_out(l, b):
            pltpu.make_async_copy(
                o_buf.at[b], out_ref(l, b), osem.at[b]).wait()

        for b in range(_NB):
            prep(b, b)
            fire_gather(b, b)

        def outer(i, carry):
            for b in range(_NB):
                l = i * _NB + b
                drain_gather(l, b)

                @pl.when(i > 0)
                def _free_obuf():
                    wait_out(l - _NB, b)

                select(b)

                @pl.when(i < niter - 1)
                def _refill():
                    prep(l + _NB, b)
                    fire_gather(l + _NB, b)

                fire_out(l, b)
            return carry

        lax.fori_loop(0, niter, outer, 0)
        for b in range(_NB):
            wait_out(blocks_per_w - _NB + b, b)

    return emb(tokg, w2)


def kernel(token_ids, weight):
    B, S = token_ids.shape
    V, D = weight.shape
    tokg = jnp.transpose(token_ids).reshape(S * (B // 128), 128)
    tokg = tokg.astype(jnp.int32)
    w2 = weight.reshape(V // 2, 2 * D)
    out3d = _embed_sc(tokg, w2, B, S, D)
    return jnp.transpose(out3d, (2, 0, 1))


# speedup vs baseline: 1.9002x; 1.9002x over previous
import functools

import jax
import jax.numpy as jnp
from jax import lax
from jax.experimental import pallas as pl
from jax.experimental.pallas import tpu as pltpu
from jax.experimental.pallas import tpu_sc as plsc

_NC = 2
_NS = 16
_NW = _NC * _NS
_NB = 2


def _embed_sc(tokg, w2, B, S, D):
    G = tokg.shape[0]
    blocks_per_w = G // _NW
    niter = blocks_per_w // _NB

    mesh = plsc.VectorSubcoreMesh(core_axis_name="c", subcore_axis_name="s")

    @functools.partial(
        pl.kernel,
        mesh=mesh,
        compiler_params=pltpu.CompilerParams(needs_layout_passes=False),
        out_type=jax.ShapeDtypeStruct((S, D, B), jnp.float32),
        scratch_types=[
            pltpu.VMEM((blocks_per_w, 128), jnp.int32),
            pltpu.VMEM((_NB, 128), jnp.int32),
            pltpu.VMEM((_NB, 128), jnp.int32),
            pltpu.VMEM((_NB, 128, 128), jnp.float32),
            pltpu.VMEM((_NB, D, 128), jnp.float32),
            pltpu.SemaphoreType.DMA((_NB,)),
            pltpu.SemaphoreType.DMA((_NB,)),
        ],
    )
    def emb(tok_hbm, w_hbm, out_hbm, idx_all, pr_idx, off_all, g_pair,
            o_buf, gsem, osem):
        wid = lax.axis_index("s") * _NC + lax.axis_index("c")
        base = wid * blocks_per_w

        pltpu.sync_copy(tok_hbm.at[pl.ds(base, blocks_per_w)], idx_all)

        def prep(l, b):
            for v in range(8):
                tid = idx_all[l, pl.ds(16 * v, 16)]
                pr_idx[b, pl.ds(16 * v, 16)] = tid >> 1
                off_all[b, pl.ds(16 * v, 16)] = (tid & 1) * 64

        def fire_gather(l, b):
            pltpu.async_copy(
                w_hbm.at[pr_idx.at[b]], g_pair.at[b], gsem.at[b])

        def drain_gather(l, b):
            pltpu.make_async_copy(
                w_hbm.at[pr_idx.at[b]], g_pair.at[b], gsem.at[b]).wait()

        def select(b):
            iot = lax.iota(jnp.int32, 16)
            rows = [iot + (16 * j) for j in range(8)]
            offs = [off_all[b, pl.ds(16 * j, 16)] for j in range(8)]
            def dblk(i0, carry):
                for dd in range(8):
                    dcol = ((8 * i0 + dd) + iot) & 63
                    for j in range(8):
                        val = plsc.load_gather(
                            g_pair.at[b], [rows[j], offs[j] + dcol])
                        plsc.store_scatter(
                            o_buf.at[b], [dcol, rows[j]], val)
                return carry

            lax.fori_loop(0, D // 8, dblk, 0)

        def out_ref(l, b):
            g = base + l
            s_idx = g >> 7
            bg = g & 127
            return out_hbm.at[s_idx, :, pl.ds(bg * 128, 128)]

        def fire_out(l, b):
            pltpu.async_copy(o_buf.at[b], out_ref(l, b), osem.at[b])

        def wait_out(l, b):
            pltpu.make_async_copy(
                o_buf.at[b], out_ref(l, b), osem.at[b]).wait()

        for b in range(_NB):
            prep(b, b)
            fire_gather(b, b)

        def outer(i, carry):
            for b in range(_NB):
                l = i * _NB + b
                drain_gather(l, b)

                @pl.when(i > 0)
                def _free_obuf():
                    wait_out(l - _NB, b)

                select(b)

                @pl.when(i < niter - 1)
                def _refill():
                    prep(l + _NB, b)
                    fire_gather(l + _NB, b)

                fire_out(l, b)
            return carry

        lax.fori_loop(0, niter, outer, 0)
        for b in range(_NB):
            wait_out(blocks_per_w - _NB + b, b)

    return emb(tokg, w2)


def kernel(token_ids, weight):
    B, S = token_ids.shape
    V, D = weight.shape
    tokg = jnp.transpose(token_ids).reshape(S * (B // 128), 128)
    tokg = tokg.astype(jnp.int32)
    w2 = weight.reshape(V // 2, 2 * D)
    out3d = _embed_sc(tokg, w2, B, S, D)
    return jnp.transpose(out3d, (2, 0, 1))
